# Initial kernel scaffold; baseline (speedup 1.0000x reference)
import jax
import jax.numpy as jnp
from jax.experimental import pallas as pl


def kernel(x, fc1_w, fc1_b, bn1_g, bn1_b, gc_w, gc_b, gc_bn_g, gc_bn_b, fc2_w, fc2_b, bn2_g, bn2_b):
    raise NotImplementedError("write your pallas kernel here")


# speedup vs baseline: 3.4157x; 3.4157x over previous
import jax
import jax.numpy as jnp
from jax import lax
from jax.experimental import pallas as pl

WS = 8
KNN = 9
EPS_BN = 1e-5
NPW = WS * WS
GW = 8
GSZ = GW * NPW

_F32 = jnp.float32
_HI = lax.Precision.HIGHEST


def _dot(a, b, dims):
    return lax.dot_general(a, b, (dims, ((), ())),
                           preferred_element_type=_F32, precision=_HI)


def _body(xw_ref, w1_ref, b1_ref, wa_ref, wb_ref, bg_ref, w2_ref, b2_ref,
          out_ref):
    xb = xw_ref[...]
    c2 = wa_ref.shape[0]

    y = _dot(w1_ref[...], xb, ((1,), (0,))) + b1_ref[...]

    ss = jnp.sum(y * y, axis=0, keepdims=True)
    inv = 1.0 / jnp.maximum(jnp.sqrt(ss), 1e-12)
    xn = y * inv
    sq = jnp.sum(xn * xn, axis=0, keepdims=True)

    af = _dot(wa_ref[...], y, ((1,), (0,))) + bg_ref[...]
    bf = _dot(wb_ref[...], y, ((1,), (0,)))

    colid = lax.broadcasted_iota(jnp.int32, (NPW, NPW), 1)
    outs = []
    for g in range(GW):
        sl = slice(g * NPW, (g + 1) * NPW)
        p = xn[:, sl]
        gm = _dot(p, p, ((0,), (0,)))
        sqg = sq[:, sl]
        dw = jnp.transpose(sqg) + sqg - 2.0 * gm
        bg_blk = bf[:, sl]
        m = jnp.full((c2, NPW), -jnp.inf, _F32)
        for _ in range(KNN):
            rowmin = jnp.min(dw, axis=1, keepdims=True)
            first = jnp.min(jnp.where(dw == rowmin, colid, NPW),
                            axis=1, keepdims=True)
            onehot = (colid == first).astype(_F32)
            gth = _dot(bg_blk, onehot, ((1,), (1,)))
            m = jnp.maximum(m, gth)
            dw = jnp.where(onehot > 0.5, jnp.inf, dw)
        outs.append(jnp.maximum(af[:, sl] + m, 0.0))

    e = jnp.concatenate(outs, axis=1)
    out = _dot(w2_ref[...], e, ((1,), (0,))) + b2_ref[...] + xb
    out_ref[...] = out


def kernel(x, fc1_w, fc1_b, bn1_g, bn1_b, gc_w, gc_b, gc_bn_g, gc_bn_b,
           fc2_w, fc2_b, bn2_g, bn2_b):
    b, c, h, w = x.shape
    nwh, nww = h // WS, w // WS
    tot = b * nwh * nww * NPW

    r = 1.0 / jnp.sqrt(jnp.float32(1.0 + EPS_BN))
    s1 = bn1_g * r
    w1 = fc1_w * s1[:, None]
    b1 = fc1_b * s1 + bn1_b
    sg = gc_bn_g * r
    wg = gc_w * sg[:, None]
    bgv = gc_b * sg + gc_bn_b
    wa = wg[:, :c] - wg[:, c:]
    wb = wg[:, c:]
    s2 = bn2_g * r
    w2 = fc2_w * s2[:, None]
    b2 = fc2_b * s2 + bn2_b

    xw = x.reshape(b, c, nwh, WS, nww, WS)
    xw = jnp.transpose(xw, (1, 0, 2, 4, 3, 5)).reshape(c, tot)

    out = pl.pallas_call(
        _body,
        grid=(tot // GSZ,),
        in_specs=[
            pl.BlockSpec((c, GSZ), lambda i: (0, i)),
            pl.BlockSpec((c, c), lambda i: (0, 0)),
            pl.BlockSpec((c, 1), lambda i: (0, 0)),
            pl.BlockSpec((2 * c, c), lambda i: (0, 0)),
            pl.BlockSpec((2 * c, c), lambda i: (0, 0)),
            pl.BlockSpec((2 * c, 1), lambda i: (0, 0)),
            pl.BlockSpec((c, 2 * c), lambda i: (0, 0)),
            pl.BlockSpec((c, 1), lambda i: (0, 0)),
        ],
        out_specs=pl.BlockSpec((c, GSZ), lambda i: (0, i)),
        out_shape=jax.ShapeDtypeStruct((c, tot), _F32),
    )(xw, w1, b1[:, None], wa, wb, bgv[:, None], w2, b2[:, None])

    o = out.reshape(c, b, nwh, nww, WS, WS)
    o = jnp.transpose(o, (1, 0, 2, 4, 3, 5)).reshape(b, c, h, w)
    return o


# speedup vs baseline: 3.5446x; 1.0377x over previous
import jax
import jax.numpy as jnp
from jax import lax
from jax.experimental import pallas as pl

WS = 8
KNN = 9
EPS_BN = 1e-5
NPW = WS * WS
GW = 8
GSZ = GW * NPW

_F32 = jnp.float32
_HI = lax.Precision.HIGHEST
_RBN = float(1.0 / (1.0 + EPS_BN) ** 0.5)


def _dot(a, b, dims, precision=_HI):
    return lax.dot_general(a, b, (dims, ((), ())),
                           preferred_element_type=_F32, precision=precision)


def _body(xw_ref, w1_ref, b1_ref, g1_ref, be1_ref, wa_ref, wb_ref, bg_ref,
          w2_ref, b2_ref, out_ref):
    xb = xw_ref[...]
    c2 = wa_ref.shape[0]

    y = _dot(w1_ref[...], xb, ((1,), (0,)), precision=None)
    y = (y + b1_ref[...]) * _RBN * g1_ref[...] + be1_ref[...]

    ss = jnp.sum(y * y, axis=0, keepdims=True)
    inv = 1.0 / jnp.maximum(jnp.sqrt(ss), 1e-12)
    xn = y * inv
    sq = jnp.sum(xn * xn, axis=0, keepdims=True)

    af = _dot(wa_ref[...], y, ((1,), (0,))) + bg_ref[...]
    bf = _dot(wb_ref[...], y, ((1,), (0,)))

    colid = lax.broadcasted_iota(jnp.int32, (NPW, NPW), 1)
    outs = []
    for g in range(GW):
        sl = slice(g * NPW, (g + 1) * NPW)
        p = xn[:, sl]
        gm = _dot(p, p, ((0,), (0,)), precision=None)
        sqg = sq[:, sl]
        inner = -2.0 * gm
        dw = (jnp.transpose(sqg) + inner) + sqg
        bg_blk = bf[:, sl]
        m = jnp.full((c2, NPW), -jnp.inf, _F32)
        for _ in range(KNN):
            rowmin = jnp.min(dw, axis=1, keepdims=True)
            first = jnp.min(jnp.where(dw == rowmin, colid, NPW),
                            axis=1, keepdims=True)
            onehot = (colid == first).astype(_F32)
            gth = _dot(bg_blk, onehot, ((1,), (1,)))
            m = jnp.maximum(m, gth)
            dw = jnp.where(onehot > 0.5, jnp.inf, dw)
        outs.append(jnp.maximum(af[:, sl] + m, 0.0))

    e = jnp.concatenate(outs, axis=1)
    out = _dot(w2_ref[...], e, ((1,), (0,))) + b2_ref[...] + xb
    out_ref[...] = out


def kernel(x, fc1_w, fc1_b, bn1_g, bn1_b, gc_w, gc_b, gc_bn_g, gc_bn_b,
           fc2_w, fc2_b, bn2_g, bn2_b):
    b, c, h, w = x.shape
    nwh, nww = h // WS, w // WS
    tot = b * nwh * nww * NPW

    r = 1.0 / jnp.sqrt(jnp.float32(1.0 + EPS_BN))
    sg = gc_bn_g * r
    wg = gc_w * sg[:, None]
    bgv = gc_b * sg + gc_bn_b
    wa = wg[:, :c] - wg[:, c:]
    wb = wg[:, c:]
    s2 = bn2_g * r
    w2 = fc2_w * s2[:, None]
    b2 = fc2_b * s2 + bn2_b

    xw = x.reshape(b, c, nwh, WS, nww, WS)
    xw = jnp.transpose(xw, (1, 0, 2, 4, 3, 5)).reshape(c, tot)

    out = pl.pallas_call(
        _body,
        grid=(tot // GSZ,),
        in_specs=[
            pl.BlockSpec((c, GSZ), lambda i: (0, i)),
            pl.BlockSpec((c, c), lambda i: (0, 0)),
            pl.BlockSpec((c, 1), lambda i: (0, 0)),
            pl.BlockSpec((c, 1), lambda i: (0, 0)),
            pl.BlockSpec((c, 1), lambda i: (0, 0)),
            pl.BlockSpec((2 * c, c), lambda i: (0, 0)),
            pl.BlockSpec((2 * c, c), lambda i: (0, 0)),
            pl.BlockSpec((2 * c, 1), lambda i: (0, 0)),
            pl.BlockSpec((c, 2 * c), lambda i: (0, 0)),
            pl.BlockSpec((c, 1), lambda i: (0, 0)),
        ],
        out_specs=pl.BlockSpec((c, GSZ), lambda i: (0, i)),
        out_shape=jax.ShapeDtypeStruct((c, tot), _F32),
    )(xw, fc1_w, fc1_b[:, None], bn1_g[:, None], bn1_b[:, None],
      wa, wb, bgv[:, None], w2, b2[:, None])

    o = out.reshape(c, b, nwh, nww, WS, WS)
    o = jnp.transpose(o, (1, 0, 2, 4, 3, 5)).reshape(b, c, h, w)
    return o


# speedup vs baseline: 17.2707x; 4.8724x over previous
import jax
import jax.numpy as jnp
from jax import lax
from jax.experimental import pallas as pl

WS = 8
KNN = 9
EPS_BN = 1e-5
NPW = WS * WS
GW = 8
GSZ = GW * NPW

_F32 = jnp.float32
_HI = lax.Precision.HIGHEST
_RBN = float(1.0 / (1.0 + EPS_BN) ** 0.5)


def _dot(a, b, dims, precision=_HI):
    return lax.dot_general(a, b, (dims, ((), ())),
                           preferred_element_type=_F32, precision=precision)


def _body(xw_ref, w1_ref, b1_ref, g1_ref, be1_ref, wa_ref, wb_ref, bg_ref,
          w2_ref, b2_ref, out_ref):
    xb = xw_ref[...]
    c2 = wa_ref.shape[0]

    y = _dot(w1_ref[...], xb, ((1,), (0,)), precision=None)
    y = (y + b1_ref[...]) * _RBN * g1_ref[...] + be1_ref[...]

    ss = jnp.sum(y * y, axis=0, keepdims=True)
    inv = 1.0 / jnp.maximum(jnp.sqrt(ss), 1e-12)
    xn = y * inv
    sq = jnp.sum(xn * xn, axis=0, keepdims=True)

    af = _dot(wa_ref[...], y, ((1,), (0,)), precision=None) + bg_ref[...]
    bf = _dot(wb_ref[...], y, ((1,), (0,)), precision=None)

    dts = []
    for g in range(GW):
        sl = slice(g * NPW, (g + 1) * NPW)
        p = xn[:, sl]
        gm = _dot(p, p, ((0,), (0,)), precision=None)
        sqg = sq[:, sl]
        dts.append((sqg + (-2.0 * gm)) + jnp.transpose(sqg))
    dwt = jnp.concatenate(dts, axis=1)

    rowid = lax.broadcasted_iota(jnp.int32, (NPW, GSZ), 0)
    m = None
    for _ in range(KNN):
        cmin = jnp.min(dwt, axis=0, keepdims=True)
        first = jnp.min(jnp.where(dwt == cmin, rowid, NPW),
                        axis=0, keepdims=True)
        onehot = (rowid == first).astype(_F32)
        gths = []
        for g in range(GW):
            sl = slice(g * NPW, (g + 1) * NPW)
            gths.append(_dot(bf[:, sl], onehot[:, sl], ((1,), (0,)),
                             precision=None))
        gth = jnp.concatenate(gths, axis=1)
        m = gth if m is None else jnp.maximum(m, gth)
        dwt = jnp.where(onehot > 0.5, jnp.inf, dwt)

    e = jnp.maximum(af + m, 0.0)
    out = _dot(w2_ref[...], e, ((1,), (0,)), precision=None) \
        + b2_ref[...] + xb
    out_ref[...] = out


def kernel(x, fc1_w, fc1_b, bn1_g, bn1_b, gc_w, gc_b, gc_bn_g, gc_bn_b,
           fc2_w, fc2_b, bn2_g, bn2_b):
    b, c, h, w = x.shape
    nwh, nww = h // WS, w // WS
    tot = b * nwh * nww * NPW

    r = 1.0 / jnp.sqrt(jnp.float32(1.0 + EPS_BN))
    sg = gc_bn_g * r
    wg = gc_w * sg[:, None]
    bgv = gc_b * sg + gc_bn_b
    wa = wg[:, :c] - wg[:, c:]
    wb = wg[:, c:]
    s2 = bn2_g * r
    w2 = fc2_w * s2[:, None]
    b2 = fc2_b * s2 + bn2_b

    xw = x.reshape(b, c, nwh, WS, nww, WS)
    xw = jnp.transpose(xw, (1, 0, 2, 4, 3, 5)).reshape(c, tot)

    out = pl.pallas_call(
        _body,
        grid=(tot // GSZ,),
        in_specs=[
            pl.BlockSpec((c, GSZ), lambda i: (0, i)),
            pl.BlockSpec((c, c), lambda i: (0, 0)),
            pl.BlockSpec((c, 1), lambda i: (0, 0)),
            pl.BlockSpec((c, 1), lambda i: (0, 0)),
            pl.BlockSpec((c, 1), lambda i: (0, 0)),
            pl.BlockSpec((2 * c, c), lambda i: (0, 0)),
            pl.BlockSpec((2 * c, c), lambda i: (0, 0)),
            pl.BlockSpec((2 * c, 1), lambda i: (0, 0)),
            pl.BlockSpec((c, 2 * c), lambda i: (0, 0)),
            pl.BlockSpec((c, 1), lambda i: (0, 0)),
        ],
        out_specs=pl.BlockSpec((c, GSZ), lambda i: (0, i)),
        out_shape=jax.ShapeDtypeStruct((c, tot), _F32),
    )(xw, fc1_w, fc1_b[:, None], bn1_g[:, None], bn1_b[:, None],
      wa, wb, bgv[:, None], w2, b2[:, None])

    o = out.reshape(c, b, nwh, nww, WS, WS)
    o = jnp.transpose(o, (1, 0, 2, 4, 3, 5)).reshape(b, c, h, w)
    return o
